# Initial kernel scaffold; baseline (speedup 1.0000x reference)
import jax
import jax.numpy as jnp
from jax.experimental import pallas as pl


def kernel(x, edge_index, edge_attr, W1, b1, W2, b2):
    raise NotImplementedError("write your pallas kernel here")


# speedup vs baseline: 6.2771x; 6.2771x over previous
import functools

import jax
import jax.numpy as jnp
from jax import lax
from jax.experimental import pallas as pl
from jax.experimental.pallas import tpu as pltpu
from jax.experimental.pallas import tpu_sc as plsc

N_NODES = 10000
N_EDGES = 320000
D_NODE = 128
D_EDGE = 16
HIDDEN = 128
D_OUT = 128

NC = 2
NS = 16
L = 16
NW = NC * NS

EPT = N_EDGES // NW
S = 80
GPT = EPT // S
CH = 2000
GPC = CH // S
NCH = EPT // CH
N_PAD = 10240
NPT = N_PAD // NS


def _sc_body(col_hbm, ea_hbm, psums_hbm, pcounts_hbm,
             idx_v, chunk_v, counts_v, stage_v, acc_sh):
    cid = lax.axis_index("c")
    sid = lax.axis_index("s")
    wid = cid * NS + sid

    pltpu.sync_copy(col_hbm.at[wid], idx_v)

    zero_row = jnp.zeros((L,), jnp.float32)

    def z_stage(i, c):
        stage_v[i, :] = zero_row
        return c
    lax.fori_loop(0, NPT, z_stage, 0)

    def z_counts(i, c):
        counts_v[pl.ds(i * L, L)] = zero_row
        return c
    lax.fori_loop(0, N_NODES // L, z_counts, 0)

    pltpu.sync_copy(stage_v, acc_sh.at[pl.ds(sid * NPT, NPT)])
    plsc.subcore_barrier()

    ones = jnp.ones((L,), jnp.float32)

    def chunk_body(c, carry):
        base = wid * EPT + c * CH
        pltpu.sync_copy(ea_hbm.at[pl.ds(base, CH)], chunk_v)

        def grp_body(j, cc):
            pltpu.sync_copy(chunk_v.at[pl.ds(j * S, S)],
                            acc_sh.at[idx_v.at[c * GPC + j]],
                            add=True)
            return cc
        lax.fori_loop(0, GPC, grp_body, 0)

        def hist_row(r_local, cc):
            r = c * GPC + r_local

            def hist_sub(k, ccc):
                idx16 = idx_v[r, pl.ds(k * L, L)]
                plsc.addupdate_scatter(counts_v, [idx16], ones)
                return ccc
            lax.fori_loop(0, S // L, hist_sub, 0)
            return cc
        lax.fori_loop(0, GPC, hist_row, 0)
        return carry
    lax.fori_loop(0, NCH, chunk_body, 0)

    plsc.subcore_barrier()

    pltpu.sync_copy(acc_sh.at[pl.ds(sid * NPT, NPT)], stage_v)
    pltpu.sync_copy(stage_v, psums_hbm.at[cid, sid])
    pltpu.sync_copy(counts_v, pcounts_hbm.at[wid, 0])


_sc_scatter = functools.partial(
    pl.kernel,
    out_type=[
        jax.ShapeDtypeStruct((NC, NS, NPT, D_EDGE), jnp.float32),
        jax.ShapeDtypeStruct((NW, 1, N_NODES), jnp.float32),
    ],
    mesh=plsc.VectorSubcoreMesh(core_axis_name="c", subcore_axis_name="s",
                                num_cores=NC, num_subcores=NS),
    compiler_params=pltpu.CompilerParams(needs_layout_passes=False,
                                         use_tc_tiling_on_sc=False),
    scratch_types=[
        pltpu.VMEM((GPT, S), jnp.int32),
        pltpu.VMEM((CH, D_EDGE), jnp.float32),
        pltpu.VMEM((N_NODES,), jnp.float32),
        pltpu.VMEM((NPT, D_EDGE), jnp.float32),
        pltpu.VMEM_SHARED((N_PAD, D_EDGE), jnp.float32),
    ],
)(_sc_body)


def _mlp_body(x_ref, ps_ref, pc_ref, w1_ref, b1_ref, w2_ref, b2_ref, o_ref):
    sums = ps_ref[0] + ps_ref[1]
    counts = jnp.sum(pc_ref[...], axis=0)
    mean = sums / jnp.maximum(counts, 1.0)[:, None]
    h = jnp.dot(x_ref[...], w1_ref[:D_NODE, :],
                preferred_element_type=jnp.float32,
                precision=lax.Precision.HIGHEST)
    h += jnp.dot(mean, w1_ref[D_NODE:, :],
                 preferred_element_type=jnp.float32,
                 precision=lax.Precision.HIGHEST)
    h = jnp.maximum(h + b1_ref[...], 0.0)
    o_ref[...] = jnp.dot(h, w2_ref[...],
                         preferred_element_type=jnp.float32,
                         precision=lax.Precision.HIGHEST) + b2_ref[...]


_B = 1024

_mlp = pl.pallas_call(
    _mlp_body,
    out_shape=jax.ShapeDtypeStruct((N_NODES, D_OUT), jnp.float32),
    grid=(pl.cdiv(N_NODES, _B),),
    in_specs=[
        pl.BlockSpec((_B, D_NODE), lambda i: (i, 0)),
        pl.BlockSpec((NC, _B, D_EDGE), lambda i: (0, i, 0)),
        pl.BlockSpec((NW, _B), lambda i: (0, i)),
        pl.BlockSpec((D_NODE + D_EDGE, HIDDEN), lambda i: (0, 0)),
        pl.BlockSpec((1, HIDDEN), lambda i: (0, 0)),
        pl.BlockSpec((HIDDEN, D_OUT), lambda i: (0, 0)),
        pl.BlockSpec((1, D_OUT), lambda i: (0, 0)),
    ],
    out_specs=pl.BlockSpec((_B, D_OUT), lambda i: (i, 0)),
)


@jax.jit
def kernel(x, edge_index, edge_attr, W1, b1, W2, b2):
    col3d = edge_index[1].reshape(NW, GPT, S)
    psums, pcounts = _sc_scatter(col3d, edge_attr)
    psums = psums.reshape(NC, N_PAD, D_EDGE)
    pcounts = pcounts.reshape(NW, N_NODES)
    return _mlp(x, psums, pcounts, W1,
                b1.reshape(1, HIDDEN), W2, b2.reshape(1, D_OUT))


# speedup vs baseline: 6.8298x; 1.0880x over previous
import functools

import jax
import jax.numpy as jnp
from jax import lax
from jax.experimental import pallas as pl
from jax.experimental.pallas import tpu as pltpu
from jax.experimental.pallas import tpu_sc as plsc

N_NODES = 10000
N_EDGES = 320000
D_NODE = 128
D_EDGE = 16
HIDDEN = 128
D_OUT = 128

NC = 2
NS = 16
L = 16
NW = NC * NS

S = 128
NG = N_EDGES // S
GBASE = NG // NW
GEXTRA = NG - GBASE * NW
GPC = 13
NCH = GBASE // GPC
CHE = GPC * S
IDXW = 88
NGPAD = 2504
N_PAD = 10240
NPT = N_PAD // NS
CR = N_PAD // S


def _sc_body(col_hbm, ea_hbm, psums_hbm, pcounts_hbm,
             idx_v, chunk0_v, chunk1_v, counts_v, stage_v, acc_sh,
             isem, csem0, csem1, ssem):
    cid = lax.axis_index("c")
    sid = lax.axis_index("s")
    wid = cid * NS + sid

    g0 = GBASE * wid + jnp.minimum(wid, GEXTRA)
    galign = (g0 >> 3) << 3
    goff = g0 - galign

    d_idx = pltpu.async_copy(col_hbm.at[pl.ds(galign, IDXW)], idx_v, isem)
    bufs = (chunk0_v, chunk1_v)
    sems = (csem0, csem1)
    chunk_descs = [None, None]
    chunk_descs[0] = pltpu.async_copy(
        ea_hbm.at[pl.ds(g0 * S, CHE)], chunk0_v, csem0)

    zero_row = jnp.zeros((L,), jnp.float32)

    def z_stage(i, c):
        stage_v[i, :] = zero_row
        return c
    lax.fori_loop(0, NPT, z_stage, 0)

    def z_counts(i, c):
        counts_v[i >> 3, pl.ds((i & 7) * L, L)] = zero_row
        return c
    lax.fori_loop(0, CR * 8, z_counts, 0)

    pltpu.sync_copy(stage_v, acc_sh.at[pl.ds(sid * NPT, NPT)])

    d_idx.wait()
    plsc.subcore_barrier()

    ones = jnp.ones((L,), jnp.float32)

    def hist_group(r):
        for k in range(S // L):
            idx16 = idx_v[r, pl.ds(k * L, L)]
            plsc.addupdate_scatter(
                counts_v, [idx16 >> 7, idx16 & 127], ones)

    for c in range(NCH):
        if c + 1 < NCH:
            chunk_descs[(c + 1) % 2] = pltpu.async_copy(
                ea_hbm.at[pl.ds((g0 + (c + 1) * GPC) * S, CHE)],
                bufs[(c + 1) % 2], sems[(c + 1) % 2])
        chunk_descs[c % 2].wait()
        buf = bufs[c % 2]
        sc_descs = []
        for j in range(GPC):
            r = goff + c * GPC + j
            sc_descs.append(pltpu.async_copy(
                buf.at[pl.ds(j * S, S)], acc_sh.at[idx_v.at[r]],
                ssem, add=True))

        def hist_body(j, cc):
            hist_group(goff + c * GPC + j)
            return cc
        lax.fori_loop(0, GPC, hist_body, 0)
        for d in sc_descs:
            d.wait()

    @pl.when(wid < GEXTRA)
    def _tail():
        pltpu.sync_copy(ea_hbm.at[pl.ds((g0 + GBASE) * S, S)], chunk0_v.at[pl.ds(0, S)])
        r = goff + GBASE
        pltpu.sync_copy(chunk0_v.at[pl.ds(0, S)], acc_sh.at[idx_v.at[r]],
                        add=True)
        hist_group(r)

    plsc.subcore_barrier()

    pltpu.sync_copy(acc_sh.at[pl.ds(sid * NPT, NPT)], stage_v)
    pltpu.sync_copy(stage_v, psums_hbm.at[cid, sid])
    pltpu.sync_copy(counts_v, pcounts_hbm.at[wid])


_sc_scatter = functools.partial(
    pl.kernel,
    out_type=[
        jax.ShapeDtypeStruct((NC, NS, NPT, D_EDGE), jnp.float32),
        jax.ShapeDtypeStruct((NW, CR, S), jnp.float32),
    ],
    mesh=plsc.VectorSubcoreMesh(core_axis_name="c", subcore_axis_name="s",
                                num_cores=NC, num_subcores=NS),
    compiler_params=pltpu.CompilerParams(needs_layout_passes=False,
                                         use_tc_tiling_on_sc=False),
    scratch_types=[
        pltpu.VMEM((IDXW, S), jnp.int32),
        pltpu.VMEM((CHE, D_EDGE), jnp.float32),
        pltpu.VMEM((CHE, D_EDGE), jnp.float32),
        pltpu.VMEM((CR, S), jnp.float32),
        pltpu.VMEM((NPT, D_EDGE), jnp.float32),
        pltpu.VMEM_SHARED((N_PAD, D_EDGE), jnp.float32),
        pltpu.SemaphoreType.DMA,
        pltpu.SemaphoreType.DMA,
        pltpu.SemaphoreType.DMA,
        pltpu.SemaphoreType.DMA,
    ],
)(_sc_body)


def _mlp_body(x_ref, ps_ref, pc_ref, w1_ref, b1_ref, w2_ref, b2_ref, o_ref):
    sums = ps_ref[0] + ps_ref[1]
    counts = jnp.sum(pc_ref[...], axis=0)
    mean = sums / jnp.maximum(counts, 1.0)[:, None]
    h = jnp.dot(x_ref[...], w1_ref[:D_NODE, :],
                preferred_element_type=jnp.float32,
                precision=lax.Precision.HIGHEST)
    h += jnp.dot(mean, w1_ref[D_NODE:, :],
                 preferred_element_type=jnp.float32,
                 precision=lax.Precision.HIGHEST)
    h = jnp.maximum(h + b1_ref[...], 0.0)
    o_ref[...] = jnp.dot(h, w2_ref[...],
                         preferred_element_type=jnp.float32,
                         precision=lax.Precision.HIGHEST) + b2_ref[...]


_B = 1024

_mlp = pl.pallas_call(
    _mlp_body,
    out_shape=jax.ShapeDtypeStruct((N_NODES, D_OUT), jnp.float32),
    grid=(pl.cdiv(N_NODES, _B),),
    in_specs=[
        pl.BlockSpec((_B, D_NODE), lambda i: (i, 0)),
        pl.BlockSpec((NC, _B, D_EDGE), lambda i: (0, i, 0)),
        pl.BlockSpec((NW, _B), lambda i: (0, i)),
        pl.BlockSpec((D_NODE + D_EDGE, HIDDEN), lambda i: (0, 0)),
        pl.BlockSpec((1, HIDDEN), lambda i: (0, 0)),
        pl.BlockSpec((HIDDEN, D_OUT), lambda i: (0, 0)),
        pl.BlockSpec((1, D_OUT), lambda i: (0, 0)),
    ],
    out_specs=pl.BlockSpec((_B, D_OUT), lambda i: (i, 0)),
)


@jax.jit
def kernel(x, edge_index, edge_attr, W1, b1, W2, b2):
    col2d = edge_index[1].reshape(NG, S)
    col2d = jnp.pad(col2d, ((0, NGPAD - NG), (0, 0)))
    psums, pcounts = _sc_scatter(col2d, edge_attr)
    psums = psums.reshape(NC, N_PAD, D_EDGE)
    pcounts = pcounts.reshape(NW, N_PAD)
    return _mlp(x, psums, pcounts, W1,
                b1.reshape(1, HIDDEN), W2, b2.reshape(1, D_OUT))


# speedup vs baseline: 11.6269x; 1.7024x over previous
import functools

import jax
import jax.numpy as jnp
from jax import lax
from jax.experimental import pallas as pl
from jax.experimental.pallas import tpu as pltpu
from jax.experimental.pallas import tpu_sc as plsc

N_NODES = 10000
N_EDGES = 320000
D_NODE = 128
D_EDGE = 16
HIDDEN = 128
D_OUT = 128

NC = 2
NS = 16
L = 16
NW = NC * NS

EPH = N_EDGES // NC
CH = 16000
NCH = EPH // CH
CB = CH // 128
NB = N_EDGES // 128
VPC = CH // L
N_PAD = 10240
CR = N_PAD // 128


def _sc_body(ei_hbm, ea_hbm, psums_hbm, pcounts_hbm,
             idx0_v, idx1_v, dat0_v, dat1_v, acc_v, counts_v,
             isem0, isem1, dsem0, dsem1):
    cid = lax.axis_index("c")
    sid = lax.axis_index("s")
    wid = cid * NS + sid

    bbase = cid * (NB // NC)

    idx_bufs = (idx0_v, idx1_v)
    dat_bufs = (dat0_v, dat1_v)
    isems = (isem0, isem1)
    dsems = (dsem0, dsem1)

    rt = sid >> 3
    rr = sid & 7

    def start_chunk(k, slot):
        di = pltpu.async_copy(
            ei_hbm.at[pl.ds(bbase + k * CB, CB), 1], idx_bufs[slot],
            isems[slot])
        dd = pltpu.async_copy(
            ea_hbm.at[rt, pl.ds(bbase + k * CB, CB), rr],
            dat_bufs[slot], dsems[slot])
        return di, dd

    descs = [None, None]
    descs[0] = start_chunk(0, 0)

    zero_row = jnp.zeros((L,), jnp.float32)

    def z_acc(i, c):
        acc_v[i >> 3, pl.ds((i & 7) * L, L)] = zero_row
        counts_v[i >> 3, pl.ds((i & 7) * L, L)] = zero_row
        return c
    lax.fori_loop(0, CR * 8, z_acc, 0, unroll=8)

    ho = sid * 62 + jnp.minimum(sid, 8)
    hn = jnp.where(sid < 8, 63, 62)
    ones = jnp.ones((L,), jnp.float32)

    for k in range(NCH):
        if k + 1 < NCH:
            descs[(k + 1) % 2] = start_chunk(k + 1, (k + 1) % 2)
        di, dd = descs[k % 2]
        di.wait()
        dd.wait()
        idx_b = idx_bufs[k % 2]
        dat_b = dat_bufs[k % 2]

        def scat(m, c):
            idx16 = idx_b[m >> 3, pl.ds((m & 7) * L, L)]
            val16 = dat_b[m >> 3, pl.ds((m & 7) * L, L)]
            plsc.addupdate_scatter(
                acc_v, [idx16 >> 7, idx16 & 127], val16)
            return c
        lax.fori_loop(0, VPC, scat, 0, unroll=8)

        def hist(j, c):
            m = ho + j
            idx16 = idx_b[m >> 3, pl.ds((m & 7) * L, L)]
            mask = jnp.broadcast_to(j < hn, (L,))
            plsc.addupdate_scatter(
                counts_v, [idx16 >> 7, idx16 & 127], ones, mask=mask)
            return c
        lax.fori_loop(0, 63, hist, 0, unroll=8)

    pltpu.sync_copy(acc_v, psums_hbm.at[wid])
    pltpu.sync_copy(counts_v, pcounts_hbm.at[wid])


_sc_scatter = functools.partial(
    pl.kernel,
    out_type=[
        jax.ShapeDtypeStruct((NW, CR, 128), jnp.float32),
        jax.ShapeDtypeStruct((NW, CR, 128), jnp.float32),
    ],
    mesh=plsc.VectorSubcoreMesh(core_axis_name="c", subcore_axis_name="s",
                                num_cores=NC, num_subcores=NS),
    compiler_params=pltpu.CompilerParams(needs_layout_passes=False,
                                         use_tc_tiling_on_sc=False),
    scratch_types=[
        pltpu.VMEM((CB, 128), jnp.int32),
        pltpu.VMEM((CB, 128), jnp.int32),
        pltpu.VMEM((CB, 128), jnp.float32),
        pltpu.VMEM((CB, 128), jnp.float32),
        pltpu.VMEM((CR, 128), jnp.float32),
        pltpu.VMEM((CR, 128), jnp.float32),
        pltpu.SemaphoreType.DMA,
        pltpu.SemaphoreType.DMA,
        pltpu.SemaphoreType.DMA,
        pltpu.SemaphoreType.DMA,
    ],
)(_sc_body)


def _mlp_body(x_ref, ps_ref, pc_ref, w1_ref, b1_ref, w2_ref, b2_ref, o_ref):
    ps = ps_ref[...].reshape(NW, _B)
    sums_t = ps[:NS] + ps[NS:]
    counts = jnp.sum(pc_ref[...].reshape(NW, _B), axis=0)
    inv = 1.0 / jnp.maximum(counts, 1.0)
    mean_t = sums_t * inv[None, :]
    hm = lax.dot_general(mean_t, w1_ref[D_NODE:, :],
                         dimension_numbers=(((0,), (0,)), ((), ())),
                         preferred_element_type=jnp.float32,
                         precision=lax.Precision.HIGHEST)
    h = jnp.dot(x_ref[...], w1_ref[:D_NODE, :],
                preferred_element_type=jnp.float32,
                precision=lax.Precision.HIGHEST)
    h = jnp.maximum(h + hm + b1_ref[...], 0.0)
    o_ref[...] = jnp.dot(h, w2_ref[...],
                         preferred_element_type=jnp.float32,
                         precision=lax.Precision.HIGHEST) + b2_ref[...]


_B = 1024

_mlp = pl.pallas_call(
    _mlp_body,
    out_shape=jax.ShapeDtypeStruct((N_NODES, D_OUT), jnp.float32),
    grid=(pl.cdiv(N_NODES, _B),),
    in_specs=[
        pl.BlockSpec((_B, D_NODE), lambda i: (i, 0)),
        pl.BlockSpec((NW, _B // 128, 128), lambda i: (0, i, 0)),
        pl.BlockSpec((NW, _B // 128, 128), lambda i: (0, i, 0)),
        pl.BlockSpec((D_NODE + D_EDGE, HIDDEN), lambda i: (0, 0)),
        pl.BlockSpec((1, HIDDEN), lambda i: (0, 0)),
        pl.BlockSpec((HIDDEN, D_OUT), lambda i: (0, 0)),
        pl.BlockSpec((1, D_OUT), lambda i: (0, 0)),
    ],
    out_specs=pl.BlockSpec((_B, D_OUT), lambda i: (i, 0)),
)


@jax.jit
def kernel(x, edge_index, edge_attr, W1, b1, W2, b2):
    ei_blocks = edge_index.reshape(2, NB, 128).transpose(1, 0, 2)
    ea_blocks = edge_attr.T.reshape(2, 8, NB, 128).transpose(0, 2, 1, 3)
    psums, pcounts = _sc_scatter(ei_blocks, ea_blocks)
    return _mlp(x, psums, pcounts, W1,
                b1.reshape(1, HIDDEN), W2, b2.reshape(1, D_OUT))
